# Initial kernel scaffold; baseline (speedup 1.0000x reference)
import functools

import jax
import jax.numpy as jnp
from jax import lax
from jax.experimental import pallas as pl
from jax.experimental.pallas import tpu as pltpu
from jax.experimental.pallas import tpu_sc as plsc

N0, N1, N2 = 10000, 5000, 2000
N0P, N1P, N2P = 10240, 5120, 2048
E0, E1 = 160000, 64000
E0P, E1P = 163840, 65536
IN_DIM, PROMPT, HID, OUT = 128, 64, 128, 64
NW = 32
C = 128
ZR = 32


def _seg_sum_sc(npad, e_per_w):
    iters = e_per_w // C
    rps = npad // 16

    mesh = plsc.VectorSubcoreMesh(core_axis_name="c", subcore_axis_name="s")

    @functools.partial(
        pl.kernel,
        mesh=mesh,
        out_type=[
            jax.ShapeDtypeStruct((npad, 128), jnp.float32),
            jax.ShapeDtypeStruct((npad, 128), jnp.float32),
            jax.ShapeDtypeStruct((npad, 16), jnp.float32),
            jax.ShapeDtypeStruct((npad, 16), jnp.float32),
        ],
        scratch_types=[
            pltpu.VMEM((iters, C), jnp.int32),
            pltpu.VMEM((iters, C), jnp.int32),
            pltpu.VMEM((C, 128), jnp.float32),
            pltpu.VMEM((C, 16), jnp.float32),
            pltpu.VMEM((ZR, 128), jnp.float32),
            pltpu.VMEM((ZR, 16), jnp.float32),
            pltpu.VMEM_SHARED((npad, 128), jnp.float32),
            pltpu.VMEM_SHARED((npad, 16), jnp.float32),
            pltpu.SemaphoreType.DMA,
        ],
    )
    def k(table, srcr, dstr, z128, z16, o16,
          out0, out1, cnt0, cnt1,
          src_v, dst_v, rows_v, ones_v, zb, zb16, acc, cnt, sem):
        cid = lax.axis_index("c")
        sid = lax.axis_index("s")
        wid = cid * 16 + sid
        base = sid * rps

        pltpu.sync_copy(z128, zb)
        pltpu.sync_copy(z16, zb16)
        pltpu.sync_copy(o16, ones_v)
        pltpu.sync_copy(srcr.at[pl.ds(wid * iters, iters)], src_v)
        pltpu.sync_copy(dstr.at[pl.ds(wid * iters, iters)], dst_v)

        for b in range(rps // ZR):
            pltpu.sync_copy(zb, acc.at[pl.ds(base + b * ZR, ZR)])
            pltpu.sync_copy(zb16, cnt.at[pl.ds(base + b * ZR, ZR)])
        plsc.subcore_barrier()

        def body(i, carry):
            src_row = src_v.at[i]
            dst_row = dst_v.at[i]
            pltpu.async_copy(table.at[src_row], rows_v, sem).wait()
            pltpu.sync_copy(rows_v, acc.at[dst_row], add=True)
            pltpu.sync_copy(ones_v, cnt.at[dst_row], add=True)
            return carry

        lax.fori_loop(0, iters, body, 0)
        plsc.subcore_barrier()

        @pl.when(cid == 0)
        def _():
            for b in range(rps // ZR):
                sl = pl.ds(base + b * ZR, ZR)
                pltpu.sync_copy(acc.at[sl], zb)
                pltpu.sync_copy(zb, out0.at[sl])
                pltpu.sync_copy(cnt.at[sl], zb16)
                pltpu.sync_copy(zb16, cnt0.at[sl])

        @pl.when(cid == 1)
        def _():
            for b in range(rps // ZR):
                sl = pl.ds(base + b * ZR, ZR)
                pltpu.sync_copy(acc.at[sl], zb)
                pltpu.sync_copy(zb, out1.at[sl])
                pltpu.sync_copy(cnt.at[sl], zb16)
                pltpu.sync_copy(zb16, cnt1.at[sl])

    return k


def _full(shape):
    return pl.BlockSpec(shape, lambda i: (0, 0))


def _tc1(fp, mp, Wpi, bpi, Wpo, bpo, Wn1a, Wn1b, Ws1a, Ws1b, b1r):
    B = 512
    G = N0P // B

    def body(x_r, m_r, wpi, bpi_r, wpo, bpo_r, wn1a, wn1b, ws1a, ws1b, b1_r,
             z_r, w_r):
        x = x_r[...]
        pin = jnp.maximum(
            jnp.dot(x, wpi[...], preferred_element_type=jnp.float32) + bpi_r[...], 0.0)
        pout = jnp.maximum(
            jnp.dot(x, wpo[...], preferred_element_type=jnp.float32) + bpo_r[...], 0.0)
        sel = jnp.where(m_r[...] > 0.0, pin, pout)
        z_r[...] = (jnp.dot(x, wn1a[...], preferred_element_type=jnp.float32)
                    + jnp.dot(sel, wn1b[...], preferred_element_type=jnp.float32))
        w_r[...] = (jnp.dot(x, ws1a[...], preferred_element_type=jnp.float32)
                    + jnp.dot(sel, ws1b[...], preferred_element_type=jnp.float32)
                    + b1_r[...])

    return pl.pallas_call(
        body,
        grid=(G,),
        in_specs=[
            pl.BlockSpec((B, IN_DIM), lambda i: (i, 0)),
            pl.BlockSpec((B, 1), lambda i: (i, 0)),
            _full((IN_DIM, PROMPT)), _full((1, PROMPT)),
            _full((IN_DIM, PROMPT)), _full((1, PROMPT)),
            _full((IN_DIM, HID)), _full((PROMPT, HID)),
            _full((IN_DIM, HID)), _full((PROMPT, HID)),
            _full((1, HID)),
        ],
        out_specs=[
            pl.BlockSpec((B, HID), lambda i: (i, 0)),
            pl.BlockSpec((B, HID), lambda i: (i, 0)),
        ],
        out_shape=[
            jax.ShapeDtypeStruct((N0P, HID), jnp.float32),
            jax.ShapeDtypeStruct((N0P, HID), jnp.float32),
        ],
    )(fp, mp, Wpi, bpi, Wpo, bpo, Wn1a, Wn1b, Ws1a, Ws1b, b1r)


def _tc2(w1b, p0, p1, c0, c1, Wn2, Ws2, b2r):
    B = 512
    G = N1P // B

    def body(w_r, p0_r, p1_r, c0_r, c1_r, wn2, ws2, b2_r, z_r, o_r):
        cntv = (c0_r[...] + c1_r[...])[:, 0:1]
        agg = (p0_r[...] + p1_r[...]) / jnp.maximum(cntv, 1.0)
        h1 = jnp.maximum(w_r[...] + agg, 0.0)
        z_r[...] = jnp.dot(h1, wn2[...], preferred_element_type=jnp.float32)
        o_r[...] = (jnp.dot(h1, ws2[...], preferred_element_type=jnp.float32)
                    + b2_r[...])

    return pl.pallas_call(
        body,
        grid=(G,),
        in_specs=[
            pl.BlockSpec((B, HID), lambda i: (i, 0)),
            pl.BlockSpec((B, HID), lambda i: (i, 0)),
            pl.BlockSpec((B, HID), lambda i: (i, 0)),
            pl.BlockSpec((B, 16), lambda i: (i, 0)),
            pl.BlockSpec((B, 16), lambda i: (i, 0)),
            _full((HID, HID)), _full((HID, HID)), _full((1, HID)),
        ],
        out_specs=[
            pl.BlockSpec((B, HID), lambda i: (i, 0)),
            pl.BlockSpec((B, HID), lambda i: (i, 0)),
        ],
        out_shape=[
            jax.ShapeDtypeStruct((N1P, HID), jnp.float32),
            jax.ShapeDtypeStruct((N1P, HID), jnp.float32),
        ],
    )(w1b, p0, p1, c0, c1, Wn2, Ws2, b2r)


def _tc3(w2b, p0, p1, c0, c1, Wc, bcr):
    B = 256
    G = N2P // B

    def body(w_r, p0_r, p1_r, c0_r, c1_r, wc, bc_r, o_r):
        cntv = (c0_r[...] + c1_r[...])[:, 0:1]
        agg = (p0_r[...] + p1_r[...]) / jnp.maximum(cntv, 1.0)
        h2 = w_r[...] + agg
        o_r[...] = (jnp.dot(h2, wc[...], preferred_element_type=jnp.float32)
                    + bc_r[...])

    return pl.pallas_call(
        body,
        grid=(G,),
        in_specs=[
            pl.BlockSpec((B, HID), lambda i: (i, 0)),
            pl.BlockSpec((B, HID), lambda i: (i, 0)),
            pl.BlockSpec((B, HID), lambda i: (i, 0)),
            pl.BlockSpec((B, 16), lambda i: (i, 0)),
            pl.BlockSpec((B, 16), lambda i: (i, 0)),
            _full((HID, OUT)), _full((1, OUT)),
        ],
        out_specs=[pl.BlockSpec((B, OUT), lambda i: (i, 0))],
        out_shape=jax.ShapeDtypeStruct((N2P, OUT), jnp.float32),
    )(w2b, p0, p1, c0, c1, Wc, bcr)


@jax.jit
def kernel(features, membership_mask, block0_src, block0_dst, block1_src,
           block1_dst, output_nodes_indices, W_pin, b_pin, W_pout, b_pout,
           Ws1, Wn1, b1, Ws2, Wn2, b2, Wc, bc):
    del output_nodes_indices

    fp = jnp.pad(features, ((0, N0P - N0), (0, 0)))
    mp = jnp.pad(membership_mask.astype(jnp.float32)[:, None],
                 ((0, N0P - N0), (0, 0)))

    s0 = jnp.pad(block0_src, (0, E0P - E0), constant_values=N0P - 1).reshape(-1, C)
    d0 = jnp.pad(block0_dst, (0, E0P - E0), constant_values=N1P - 1).reshape(-1, C)
    s1 = jnp.pad(block1_src, (0, E1P - E1), constant_values=N1P - 1).reshape(-1, C)
    d1 = jnp.pad(block1_dst, (0, E1P - E1), constant_values=N2P - 1).reshape(-1, C)

    z128 = jnp.zeros((ZR, 128), jnp.float32)
    z16 = jnp.zeros((ZR, 16), jnp.float32)
    o16 = jnp.ones((C, 16), jnp.float32)

    z1, w1 = _tc1(fp, mp,
                  W_pin, b_pin[None, :], W_pout, b_pout[None, :],
                  Wn1[:IN_DIM], Wn1[IN_DIM:], Ws1[:IN_DIM], Ws1[IN_DIM:],
                  b1[None, :])

    p0, p1, c0, c1 = _seg_sum_sc(N1P, E0P // NW)(z1, s0, d0, z128, z16, o16)
    z2, w2 = _tc2(w1[:N1P], p0, p1, c0, c1, Wn2, Ws2, b2[None, :])

    q0, q1, e0, e1 = _seg_sum_sc(N2P, E1P // NW)(z2, s1, d1, z128, z16, o16)
    logits = _tc3(w2[:N2P], q0, q1, e0, e1, Wc, bc[None, :])

    return logits[:N2]


# speedup vs baseline: 3.8867x; 3.8867x over previous
import functools

import jax
import jax.numpy as jnp
from jax import lax
from jax.experimental import pallas as pl
from jax.experimental.pallas import tpu as pltpu
from jax.experimental.pallas import tpu_sc as plsc

N0, N1, N2 = 10000, 5000, 2000
N0P, N1P, N2P = 10240, 5120, 2048
E0, E1 = 160000, 64000
E0P, E1P = 163840, 65536
IN_DIM, PROMPT, HID, OUT = 128, 64, 128, 64
NW = 32
ZR = 32


def _seg_sum_sc(npad, e_per_w, C):
    iters = e_per_w // C
    rps = npad // 16

    mesh = plsc.VectorSubcoreMesh(core_axis_name="c", subcore_axis_name="s")

    @functools.partial(
        pl.kernel,
        mesh=mesh,
        out_type=[
            jax.ShapeDtypeStruct((npad, 128), jnp.float32),
            jax.ShapeDtypeStruct((npad, 128), jnp.float32),
            jax.ShapeDtypeStruct((npad, 128), jnp.float32),
            jax.ShapeDtypeStruct((npad, 128), jnp.float32),
        ],
        scratch_types=[
            pltpu.VMEM((C,), jnp.int32),
            pltpu.VMEM((C,), jnp.int32),
            pltpu.VMEM((C,), jnp.int32),
            pltpu.VMEM((C,), jnp.int32),
            pltpu.VMEM((C, 128), jnp.float32),
            pltpu.VMEM((C, 128), jnp.float32),
            pltpu.VMEM((C, 128), jnp.float32),
            pltpu.VMEM((ZR, 128), jnp.float32),
            pltpu.VMEM_SHARED((npad, 128), jnp.float32),
            pltpu.VMEM_SHARED((npad, 128), jnp.float32),
            pltpu.SemaphoreType.DMA,
            pltpu.SemaphoreType.DMA,
        ],
    )
    def k(table, srcr, dstr, z128, o128,
          out0, out1, cnt0, cnt1,
          src0, src1, dst0, dst1, rows0, rows1, ones_v, zb, acc, cnt,
          sem0, sem1):
        cid = lax.axis_index("c")
        sid = lax.axis_index("s")
        wid = cid * 16 + sid
        base = sid * rps
        base_e = wid * (iters * C)
        srcb, dstb, rowsb, semb = (src0, src1), (dst0, dst1), (rows0, rows1), (sem0, sem1)

        pltpu.sync_copy(z128, zb)
        pltpu.sync_copy(o128, ones_v)

        for b in range(rps // ZR):
            pltpu.sync_copy(zb, acc.at[pl.ds(base + b * ZR, ZR)])
            pltpu.sync_copy(zb, cnt.at[pl.ds(base + b * ZR, ZR)])
        plsc.subcore_barrier()

        pltpu.sync_copy(srcr.at[pl.ds(base_e, C)], src0)
        pltpu.sync_copy(dstr.at[pl.ds(base_e, C)], dst0)
        pltpu.async_copy(table.at[src0], rows0, sem0)

        def body(i2, carry):
            for b in range(2):
                j = i2 * 2 + b
                nb = 1 - b

                @pl.when(j + 1 < iters)
                def _():
                    pltpu.sync_copy(srcr.at[pl.ds(base_e + (j + 1) * C, C)],
                                    srcb[nb])
                    pltpu.sync_copy(dstr.at[pl.ds(base_e + (j + 1) * C, C)],
                                    dstb[nb])
                    pltpu.async_copy(table.at[srcb[nb]], rowsb[nb], semb[nb])

                pltpu.make_async_copy(table.at[pl.ds(0, C)], rowsb[b],
                                      semb[b]).wait()
                pltpu.sync_copy(rowsb[b], acc.at[dstb[b]], add=True)
                pltpu.sync_copy(ones_v, cnt.at[dstb[b]], add=True)
            return carry

        lax.fori_loop(0, iters // 2, body, 0)
        plsc.subcore_barrier()

        @pl.when(cid == 0)
        def _():
            for b in range(rps // ZR):
                sl = pl.ds(base + b * ZR, ZR)
                pltpu.sync_copy(acc.at[sl], zb)
                pltpu.sync_copy(zb, out0.at[sl])
                pltpu.sync_copy(cnt.at[sl], zb)
                pltpu.sync_copy(zb, cnt0.at[sl])

        @pl.when(cid == 1)
        def _():
            for b in range(rps // ZR):
                sl = pl.ds(base + b * ZR, ZR)
                pltpu.sync_copy(acc.at[sl], zb)
                pltpu.sync_copy(zb, out1.at[sl])
                pltpu.sync_copy(cnt.at[sl], zb)
                pltpu.sync_copy(zb, cnt1.at[sl])

    return k


def _full(shape):
    return pl.BlockSpec(shape, lambda i: (0, 0))


def _tc1(fp, mp, Wpi, bpi, Wpo, bpo, Wn1a, Wn1b, Ws1a, Ws1b, b1r):
    B = 512
    G = N0P // B

    def body(x_r, m_r, wpi, bpi_r, wpo, bpo_r, wn1a, wn1b, ws1a, ws1b, b1_r,
             z_r, w_r):
        x = x_r[...]
        pin = jnp.maximum(
            jnp.dot(x, wpi[...], preferred_element_type=jnp.float32) + bpi_r[...], 0.0)
        pout = jnp.maximum(
            jnp.dot(x, wpo[...], preferred_element_type=jnp.float32) + bpo_r[...], 0.0)
        sel = jnp.where(m_r[...] > 0.0, pin, pout)
        z_r[...] = (jnp.dot(x, wn1a[...], preferred_element_type=jnp.float32)
                    + jnp.dot(sel, wn1b[...], preferred_element_type=jnp.float32))
        w_r[...] = (jnp.dot(x, ws1a[...], preferred_element_type=jnp.float32)
                    + jnp.dot(sel, ws1b[...], preferred_element_type=jnp.float32)
                    + b1_r[...])

    return pl.pallas_call(
        body,
        grid=(G,),
        in_specs=[
            pl.BlockSpec((B, IN_DIM), lambda i: (i, 0)),
            pl.BlockSpec((B, 1), lambda i: (i, 0)),
            _full((IN_DIM, PROMPT)), _full((1, PROMPT)),
            _full((IN_DIM, PROMPT)), _full((1, PROMPT)),
            _full((IN_DIM, HID)), _full((PROMPT, HID)),
            _full((IN_DIM, HID)), _full((PROMPT, HID)),
            _full((1, HID)),
        ],
        out_specs=[
            pl.BlockSpec((B, HID), lambda i: (i, 0)),
            pl.BlockSpec((B, HID), lambda i: (i, 0)),
        ],
        out_shape=[
            jax.ShapeDtypeStruct((N0P, HID), jnp.float32),
            jax.ShapeDtypeStruct((N0P, HID), jnp.float32),
        ],
    )(fp, mp, Wpi, bpi, Wpo, bpo, Wn1a, Wn1b, Ws1a, Ws1b, b1r)


def _tc2(w1b, p0, p1, c0, c1, Wn2, Ws2, b2r):
    B = 512
    G = N1P // B

    def body(w_r, p0_r, p1_r, c0_r, c1_r, wn2, ws2, b2_r, z_r, o_r):
        cntv = (c0_r[...] + c1_r[...])[:, 0:1]
        agg = (p0_r[...] + p1_r[...]) / jnp.maximum(cntv, 1.0)
        h1 = jnp.maximum(w_r[...] + agg, 0.0)
        z_r[...] = jnp.dot(h1, wn2[...], preferred_element_type=jnp.float32)
        o_r[...] = (jnp.dot(h1, ws2[...], preferred_element_type=jnp.float32)
                    + b2_r[...])

    return pl.pallas_call(
        body,
        grid=(G,),
        in_specs=[
            pl.BlockSpec((B, HID), lambda i: (i, 0)),
            pl.BlockSpec((B, HID), lambda i: (i, 0)),
            pl.BlockSpec((B, HID), lambda i: (i, 0)),
            pl.BlockSpec((B, HID), lambda i: (i, 0)),
            pl.BlockSpec((B, HID), lambda i: (i, 0)),
            _full((HID, HID)), _full((HID, HID)), _full((1, HID)),
        ],
        out_specs=[
            pl.BlockSpec((B, HID), lambda i: (i, 0)),
            pl.BlockSpec((B, HID), lambda i: (i, 0)),
        ],
        out_shape=[
            jax.ShapeDtypeStruct((N1P, HID), jnp.float32),
            jax.ShapeDtypeStruct((N1P, HID), jnp.float32),
        ],
    )(w1b, p0, p1, c0, c1, Wn2, Ws2, b2r)


def _tc3(w2b, p0, p1, c0, c1, Wc, bcr):
    B = 256
    G = N2P // B

    def body(w_r, p0_r, p1_r, c0_r, c1_r, wc, bc_r, o_r):
        cntv = (c0_r[...] + c1_r[...])[:, 0:1]
        agg = (p0_r[...] + p1_r[...]) / jnp.maximum(cntv, 1.0)
        h2 = w_r[...] + agg
        o_r[...] = (jnp.dot(h2, wc[...], preferred_element_type=jnp.float32)
                    + bc_r[...])

    return pl.pallas_call(
        body,
        grid=(G,),
        in_specs=[
            pl.BlockSpec((B, HID), lambda i: (i, 0)),
            pl.BlockSpec((B, HID), lambda i: (i, 0)),
            pl.BlockSpec((B, HID), lambda i: (i, 0)),
            pl.BlockSpec((B, HID), lambda i: (i, 0)),
            pl.BlockSpec((B, HID), lambda i: (i, 0)),
            _full((HID, OUT)), _full((1, OUT)),
        ],
        out_specs=pl.BlockSpec((B, OUT), lambda i: (i, 0)),
        out_shape=jax.ShapeDtypeStruct((N2P, OUT), jnp.float32),
    )(w2b, p0, p1, c0, c1, Wc, bcr)


@jax.jit
def kernel(features, membership_mask, block0_src, block0_dst, block1_src,
           block1_dst, output_nodes_indices, W_pin, b_pin, W_pout, b_pout,
           Ws1, Wn1, b1, Ws2, Wn2, b2, Wc, bc):
    del output_nodes_indices

    fp = jnp.pad(features, ((0, N0P - N0), (0, 0)))
    mp = jnp.pad(membership_mask.astype(jnp.float32)[:, None],
                 ((0, N0P - N0), (0, 0)))

    s0 = jnp.pad(block0_src, (0, E0P - E0), constant_values=N0P - 1)
    d0 = jnp.pad(block0_dst, (0, E0P - E0), constant_values=N1P - 1)
    s1 = jnp.pad(block1_src, (0, E1P - E1), constant_values=N1P - 1)
    d1 = jnp.pad(block1_dst, (0, E1P - E1), constant_values=N2P - 1)

    z128 = jnp.zeros((ZR, 128), jnp.float32)

    z1, w1 = _tc1(fp, mp,
                  W_pin, b_pin[None, :], W_pout, b_pout[None, :],
                  Wn1[:IN_DIM], Wn1[IN_DIM:], Ws1[:IN_DIM], Ws1[IN_DIM:],
                  b1[None, :])

    p0, p1, c0, c1 = _seg_sum_sc(N1P, E0P // NW, 64)(
        z1, s0, d0, z128, jnp.ones((64, 128), jnp.float32))
    z2, w2 = _tc2(w1[:N1P], p0, p1, c0, c1, Wn2, Ws2, b2[None, :])

    q0, q1, e0, e1 = _seg_sum_sc(N2P, E1P // NW, 128)(
        z2, s1, d1, z128, jnp.ones((128, 128), jnp.float32))
    logits = _tc3(w2[:N2P], q0, q1, e0, e1, Wc, bc[None, :])

    return logits[:N2]
